# baseline (speedup 1.0000x reference)
import jax
import jax.numpy as jnp
from jax.experimental import pallas as pl

NUM_GROUPS = 12
KEEP = 6


def _token_kernel(x_ref, out_ref, mask_ref):
    xb = x_ref[0]
    scores = jnp.sum(jnp.abs(xb), axis=(1, 2))
    sr = scores[None, :]
    sc = scores[:, None]
    col = jax.lax.broadcasted_iota(jnp.int32, (NUM_GROUPS, NUM_GROUPS), 1)
    row = jax.lax.broadcasted_iota(jnp.int32, (NUM_GROUPS, NUM_GROUPS), 0)
    beats = (sr > sc) | ((sr == sc) & (col < row))
    rank = jnp.sum(beats.astype(jnp.int32), axis=1)
    keep = (rank < KEEP).astype(jnp.float32)
    out_ref[0] = xb * keep[:, None, None]
    mask_ref[0, 0] = keep


def kernel(x):
    B, T, C, H, W = x.shape
    G = NUM_GROUPS
    Cg = C // G
    N = B * T
    xr = x.reshape(N, G, Cg, H * W)
    gated, maskf = pl.pallas_call(
        _token_kernel,
        grid=(N,),
        in_specs=[pl.BlockSpec((1, G, Cg, H * W), lambda i: (i, 0, 0, 0))],
        out_specs=[
            pl.BlockSpec((1, G, Cg, H * W), lambda i: (i, 0, 0, 0)),
            pl.BlockSpec((1, 1, G), lambda i: (i, 0, 0)),
        ],
        out_shape=[
            jax.ShapeDtypeStruct((N, G, Cg, H * W), x.dtype),
            jax.ShapeDtypeStruct((N, 1, G), jnp.float32),
        ],
    )(xr)
    return gated.reshape(B, T, C, H, W), (maskf.reshape(B, T, G) > 0)


# speedup vs baseline: 12.2654x; 12.2654x over previous
import jax
import jax.numpy as jnp
from jax.experimental import pallas as pl

NUM_GROUPS = 12
KEEP = 6
TOKENS_PER_STEP = 4


def _pruner_kernel(x_ref, out_ref, mask_ref):
    n, hw, c = x_ref.shape
    g = NUM_GROUPS
    xb = x_ref[...]
    a = jnp.abs(xb)
    colsum = jnp.sum(a, axis=1)
    lane = jax.lax.broadcasted_iota(jnp.int32, (c, g), 0)
    grp = jax.lax.broadcasted_iota(jnp.int32, (c, g), 1)
    seg = (lane // (c // g) == grp).astype(jnp.float32)
    scores = jax.lax.dot(colsum, seg)
    sr = scores[:, None, :]
    sc = scores[:, :, None]
    col = jax.lax.broadcasted_iota(jnp.int32, (n, g, g), 2)
    row = jax.lax.broadcasted_iota(jnp.int32, (n, g, g), 1)
    beats = (sr > sc) | ((sr == sc) & (col < row))
    rank = jnp.sum(beats.astype(jnp.int32), axis=2)
    keep = (rank < KEEP).astype(jnp.float32)
    gate = jax.lax.dot(keep, seg.T)
    out_ref[...] = xb * gate[:, None, :]
    mask_ref[...] = keep[:, None, :]


def kernel(x):
    B, T, C, H, W = x.shape
    G = NUM_GROUPS
    N = B * T
    TPS = TOKENS_PER_STEP
    xt = jnp.transpose(x, (0, 1, 3, 4, 2)).reshape(N, H * W, C)
    gated, maskf = pl.pallas_call(
        _pruner_kernel,
        grid=(N // TPS,),
        in_specs=[pl.BlockSpec((TPS, H * W, C), lambda i: (i, 0, 0))],
        out_specs=[
            pl.BlockSpec((TPS, H * W, C), lambda i: (i, 0, 0)),
            pl.BlockSpec((TPS, 1, G), lambda i: (i, 0, 0)),
        ],
        out_shape=[
            jax.ShapeDtypeStruct((N, H * W, C), x.dtype),
            jax.ShapeDtypeStruct((N, 1, G), jnp.float32),
        ],
    )(xt)
    gated = jnp.transpose(gated.reshape(B, T, H, W, C), (0, 1, 4, 2, 3))
    return gated, (maskf.reshape(B, T, G) > 0)
